# baseline (speedup 1.0000x reference)
import functools

import jax
import jax.numpy as jnp
from jax import lax
from jax.experimental import pallas as pl
from jax.experimental.pallas import tpu as pltpu
from jax.experimental.pallas import tpu_sc as plsc

NUM_LAYERS = 12
PER_LAYER_DIM = 64
D = NUM_LAYERS * PER_LAYER_DIM
NBATCH = 4
SEQ = 2048
B_TOTAL = NBATCH * SEQ
NW = 32
B_PER_W = B_TOTAL // NW
CHUNK = 128
NCHUNK = B_PER_W // CHUNK
LANES = 16
SCALE = float(PER_LAYER_DIM) ** 0.5

_mesh = plsc.VectorSubcoreMesh(core_axis_name="c", subcore_axis_name="s")


@functools.partial(
    pl.kernel,
    out_type=jax.ShapeDtypeStruct((NBATCH, NUM_LAYERS, PER_LAYER_DIM, SEQ), jnp.float32),
    mesh=_mesh,
    compiler_params=pltpu.CompilerParams(needs_layout_passes=False, use_tc_tiling_on_sc=False),
    scratch_types=[
        pltpu.VMEM((NCHUNK, CHUNK), jnp.int32),
        pltpu.VMEM((CHUNK, D), jnp.float32),
        pltpu.VMEM((PER_LAYER_DIM, CHUNK), jnp.float32),
        pltpu.VMEM((PER_LAYER_DIM, CHUNK), jnp.float32),
        pltpu.SemaphoreType.DMA,
        pltpu.SemaphoreType.DMA,
        pltpu.SemaphoreType.DMA,
    ],
)
def _embed(ids_hbm, table_hbm, out_phys, idx_v, rows, st0, st1, gsem, ssem0, ssem1):
    wid = lax.axis_index("s") * 2 + lax.axis_index("c")
    bb = wid // 8
    tw = (wid % 8) * B_PER_W

    pltpu.sync_copy(ids_hbm.at[pl.ds(wid * NCHUNK, NCHUNK)], idx_v)

    itoa = lax.iota(jnp.int32, 16)

    sts = (st0, st1)
    ssems = (ssem0, ssem1)

    def transpose_layer(l, st):
        @pl.loop(0, CHUNK // 16)
        def _t(T):
            tb = T * 16

            @plsc.parallel_loop(0, 16, unroll=4)
            def _s(s):
                rotv = (itoa + s) & 15
                rowv = tb + rotv
                for dd in range(4):
                    dvec = dd * 16 + itoa
                    colv = l * PER_LAYER_DIM + dvec
                    v = plsc.load_gather(rows, [rowv, colv]) * SCALE
                    plsc.store_scatter(st, [dvec, rowv], v)

    def drain(st, sem):
        pltpu.make_async_copy(out_phys.at[0, 0, :, pl.ds(0, CHUNK)], st, sem).wait()


    @pl.loop(0, NCHUNK)
    def _chunk(c):
        pltpu.async_copy(table_hbm.at[idx_v.at[c]], rows, gsem).wait()
        tslice = pl.ds(tw + c * CHUNK, CHUNK)

        @pl.loop(0, NUM_LAYERS // 2)
        def _pair(p):
            for half in range(2):
                l = p * 2 + half

                @pl.when(p > 0)
                def _():
                    drain(sts[half], ssems[half])

                transpose_layer(l, sts[half])
                pltpu.async_copy(sts[half], out_phys.at[bb, l, :, tslice], ssems[half])

        drain(st0, ssem0)
        drain(st1, ssem1)


def kernel(input_ids, table):
    ids = input_ids.reshape(NW * NCHUNK, CHUNK).astype(jnp.int32)
    out_phys = _embed(ids, table)
    return jnp.transpose(out_phys, (0, 3, 1, 2))


# speedup vs baseline: 7.7385x; 7.7385x over previous
import functools

import jax
import jax.numpy as jnp
from jax import lax
from jax.experimental import pallas as pl
from jax.experimental.pallas import tpu as pltpu
from jax.experimental.pallas import tpu_sc as plsc

NUM_LAYERS = 12
PER_LAYER_DIM = 64
D = NUM_LAYERS * PER_LAYER_DIM
NBATCH = 4
SEQ = 2048
B_TOTAL = NBATCH * SEQ
NW = 32
B_PER_W = B_TOTAL // NW
CHUNK = 128
NCHUNK = B_PER_W // CHUNK
LANES = 16
SCALE = float(PER_LAYER_DIM) ** 0.5

_mesh = plsc.VectorSubcoreMesh(core_axis_name="c", subcore_axis_name="s")


@functools.partial(
    pl.kernel,
    out_type=jax.ShapeDtypeStruct((NBATCH, NUM_LAYERS, PER_LAYER_DIM, SEQ), jnp.float32),
    mesh=_mesh,
    compiler_params=pltpu.CompilerParams(needs_layout_passes=False),
    scratch_types=[
        pltpu.VMEM((NCHUNK, CHUNK), jnp.int32),
        pltpu.VMEM((CHUNK, D), jnp.float32),
        pltpu.VMEM((PER_LAYER_DIM, CHUNK), jnp.float32),
        pltpu.VMEM((PER_LAYER_DIM, CHUNK), jnp.float32),
        pltpu.SemaphoreType.DMA,
        pltpu.SemaphoreType.DMA,
        pltpu.SemaphoreType.DMA,
    ],
)
def _embed(ids_hbm, table_hbm, out_phys, idx_v, rows, st0, st1, gsem, ssem0, ssem1):
    wid = lax.axis_index("s") * 2 + lax.axis_index("c")
    bb = wid // 8
    tw = (wid % 8) * B_PER_W

    pltpu.sync_copy(ids_hbm.at[pl.ds(wid * NCHUNK, NCHUNK)], idx_v)

    itoa = lax.iota(jnp.int32, 16)

    sts = (st0, st1)
    ssems = (ssem0, ssem1)

    def transpose_layer(l, st):
        @pl.loop(0, CHUNK // 16)
        def _t(T):
            tb = T * 16

            @plsc.parallel_loop(0, 16, unroll=4)
            def _s(s):
                rotv = (itoa + s) & 15
                rowv = tb + rotv
                for dd in range(4):
                    dvec = dd * 16 + itoa
                    colv = l * PER_LAYER_DIM + dvec
                    v = plsc.load_gather(rows, [rowv, colv]) * SCALE
                    plsc.store_scatter(st, [dvec, rowv], v)

    def drain(st, sem):
        pltpu.make_async_copy(out_phys.at[0, 0, :, pl.ds(0, CHUNK)], st, sem).wait()


    @pl.loop(0, NCHUNK)
    def _chunk(c):
        pltpu.async_copy(table_hbm.at[idx_v.at[c]], rows, gsem).wait()
        tslice = pl.ds(tw + c * CHUNK, CHUNK)

        @pl.loop(0, NUM_LAYERS // 2)
        def _pair(p):
            for half in range(2):
                l = p * 2 + half

                @pl.when(p > 0)
                def _():
                    drain(sts[half], ssems[half])

                transpose_layer(l, sts[half])
                pltpu.async_copy(sts[half], out_phys.at[bb, l, :, tslice], ssems[half])

        drain(st0, ssem0)
        drain(st1, ssem1)


def kernel(input_ids, table):
    ids = input_ids.reshape(NW * NCHUNK, CHUNK).astype(jnp.int32)
    out_phys = _embed(ids, table)
    return jnp.transpose(out_phys, (0, 3, 1, 2))


# speedup vs baseline: 7.7622x; 1.0031x over previous
import functools

import jax
import jax.numpy as jnp
from jax import lax
from jax.experimental import pallas as pl
from jax.experimental.pallas import tpu as pltpu
from jax.experimental.pallas import tpu_sc as plsc

NUM_LAYERS = 12
PER_LAYER_DIM = 64
D = NUM_LAYERS * PER_LAYER_DIM
NBATCH = 4
SEQ = 2048
B_TOTAL = NBATCH * SEQ
NW = 32
B_PER_W = B_TOTAL // NW
CHUNK = 128
NCHUNK = B_PER_W // CHUNK
LANES = 16
SCALE = float(PER_LAYER_DIM) ** 0.5

_mesh = plsc.VectorSubcoreMesh(core_axis_name="c", subcore_axis_name="s")


@functools.partial(
    pl.kernel,
    out_type=jax.ShapeDtypeStruct((NBATCH, NUM_LAYERS, PER_LAYER_DIM, SEQ), jnp.float32),
    mesh=_mesh,
    compiler_params=pltpu.CompilerParams(needs_layout_passes=False),
    scratch_types=[
        pltpu.VMEM((NCHUNK, CHUNK), jnp.int32),
        pltpu.VMEM((CHUNK, D), jnp.float32),
        pltpu.VMEM((PER_LAYER_DIM, CHUNK), jnp.float32),
        pltpu.VMEM((PER_LAYER_DIM, CHUNK), jnp.float32),
        pltpu.SemaphoreType.DMA,
        pltpu.SemaphoreType.DMA,
        pltpu.SemaphoreType.DMA,
    ],
)
def _embed(ids_hbm, table_hbm, out_phys, idx_v, rows, st0, st1, gsem, ssem0, ssem1):
    wid = lax.axis_index("s") * 2 + lax.axis_index("c")
    bb = wid // 8
    tw = (wid % 8) * B_PER_W

    pltpu.sync_copy(ids_hbm.at[pl.ds(wid * NCHUNK, NCHUNK)], idx_v)

    itoa = lax.iota(jnp.int32, 16)

    sts = (st0, st1)
    ssems = (ssem0, ssem1)

    def transpose_layer(l, st):
        @pl.loop(0, CHUNK // 16)
        def _t(T):
            tb = T * 16

            @plsc.parallel_loop(0, 16, unroll=4)
            def _s(s):
                rotv = (itoa + s) & 15
                rowv = tb + rotv
                for dd in range(4):
                    dvec = dd * 16 + itoa
                    colv = l * PER_LAYER_DIM + dvec
                    v = plsc.load_gather(rows, [rowv, colv]) * SCALE
                    plsc.store_scatter(st, [dvec, rowv], v)

    def drain(st, sem):
        pltpu.make_async_copy(out_phys.at[0, 0, :, pl.ds(0, CHUNK)], st, sem).wait()


    @pl.loop(0, NCHUNK)
    def _chunk(c):
        g = pltpu.async_copy(table_hbm.at[idx_v.at[c]], rows, gsem)

        @pl.when(c > 0)
        def _():
            drain(st0, ssem0)
            drain(st1, ssem1)

        g.wait()
        tslice = pl.ds(tw + c * CHUNK, CHUNK)

        @pl.loop(0, NUM_LAYERS // 2)
        def _pair(p):
            for half in range(2):
                l = p * 2 + half

                @pl.when(p > 0)
                def _():
                    drain(sts[half], ssems[half])

                transpose_layer(l, sts[half])
                pltpu.async_copy(sts[half], out_phys.at[bb, l, :, tslice], ssems[half])

    drain(st0, ssem0)
    drain(st1, ssem1)


def kernel(input_ids, table):
    ids = input_ids.reshape(NW * NCHUNK, CHUNK).astype(jnp.int32)
    out_phys = _embed(ids, table)
    return jnp.transpose(out_phys, (0, 3, 1, 2))
